# Initial kernel scaffold; baseline (speedup 1.0000x reference)
import jax
import jax.numpy as jnp
from jax.experimental import pallas as pl


def kernel(x, W, gamma, beta):
    raise NotImplementedError("write your pallas kernel here")


# speedup vs baseline: 8.0408x; 8.0408x over previous
import functools

import jax
import jax.numpy as jnp
from jax import lax
from jax.experimental import pallas as pl
from jax.experimental.pallas import tpu as pltpu
from jax.experimental.pallas import tpu_sc as plsc

_B, _D, _N = 4, 64, 1024
_K = 20
_COUT = 128

_NC, _NS, _L = 2, 16, 16
_NW = _NC * _NS
_PTS = _B * _N
_PER_W = _PTS // _NW
_C = 4
_CH = _PER_W // _C


def _tc_body(x_ref, wm_ref, w2t_ref, idx_ref, y1_ref, y2_ref):
    b = pl.program_id(0)
    xb = x_ref[0]
    xt = xb.T
    g = jnp.dot(xt, xb, preferred_element_type=jnp.float32)
    sqr = jnp.sum(xb * xb, axis=0, keepdims=True)
    sqc = jnp.sum(xt * xt, axis=1, keepdims=True)
    d2 = sqc + sqr - 2.0 * g
    iota = lax.broadcasted_iota(jnp.int32, (_N, _N), 1)
    kiota = lax.broadcasted_iota(jnp.int32, (_N, _K), 1)
    inf = jnp.float32(3.0e38)
    dm = d2
    idx_mat = jnp.zeros((_N, _K), dtype=jnp.int32)
    for j in range(_K):
        rowmin = jnp.min(dm, axis=1, keepdims=True)
        amin = jnp.min(jnp.where(dm == rowmin, iota, _N),
                       axis=1, keepdims=True)
        idx_mat = jnp.where(kiota == j, amin, idx_mat)
        dm = jnp.where(iota == amin, inf, dm)
    idx_ref[0] = idx_mat + b * _N
    y1_ref[0] = jnp.dot(xt, wm_ref[...], preferred_element_type=jnp.float32)
    y2_ref[0] = jnp.dot(xt, w2t_ref[...], preferred_element_type=jnp.float32)


def _tc_stage(x, wm, w2t):
    return pl.pallas_call(
        _tc_body,
        grid=(_B,),
        in_specs=[
            pl.BlockSpec((1, _D, _N), lambda b: (b, 0, 0)),
            pl.BlockSpec((_D, _COUT), lambda b: (0, 0)),
            pl.BlockSpec((_D, _COUT), lambda b: (0, 0)),
        ],
        out_specs=[
            pl.BlockSpec((1, _N, _K), lambda b: (b, 0, 0)),
            pl.BlockSpec((1, _N, _COUT), lambda b: (b, 0, 0)),
            pl.BlockSpec((1, _N, _COUT), lambda b: (b, 0, 0)),
        ],
        out_shape=[
            jax.ShapeDtypeStruct((_B, _N, _K), jnp.int32),
            jax.ShapeDtypeStruct((_B, _N, _COUT), jnp.float32),
            jax.ShapeDtypeStruct((_B, _N, _COUT), jnp.float32),
        ],
    )(x, wm, w2t)


def _sc_stage(y2t, idx_flat, y1t, scale, beta):
    mesh = plsc.VectorSubcoreMesh(core_axis_name="c", subcore_axis_name="s")

    @functools.partial(
        pl.kernel,
        mesh=mesh,
        out_type=jax.ShapeDtypeStruct((_PTS, _COUT), jnp.float32),
        scratch_types=[
            pltpu.VMEM((_C * _K,), jnp.int32),
            pltpu.VMEM((_C * _K, _COUT), jnp.float32),
            pltpu.VMEM((_C, _COUT), jnp.float32),
            pltpu.VMEM((_C, _COUT), jnp.float32),
            pltpu.VMEM((_COUT,), jnp.float32),
            pltpu.VMEM((_COUT,), jnp.float32),
            pltpu.SemaphoreType.DMA,
        ],
    )
    def sck(y2t_hbm, idx_hbm, y1t_hbm, sc_hbm, be_hbm, out_hbm,
            idx_v, rows_v, y1_v, out_v, sc_v, be_v, sem):
        wid = lax.axis_index("s") * _NC + lax.axis_index("c")
        pltpu.sync_copy(sc_hbm, sc_v)
        pltpu.sync_copy(be_hbm, be_v)

        @pl.loop(0, _CH)
        def _chunk(ci):
            pt0 = wid * _PER_W + ci * _C
            pltpu.sync_copy(idx_hbm.at[pl.ds(pt0 * _K, _C * _K)], idx_v)
            pltpu.async_copy(y2t_hbm.at[idx_v], rows_v, sem).wait()
            pltpu.sync_copy(y1t_hbm.at[pl.ds(pt0, _C)], y1_v)
            for p in range(_C):
                for g in range(_COUT // _L):
                    sl = pl.ds(g * _L, _L)
                    m = rows_v[p * _K, sl]
                    for j in range(1, _K):
                        m = jnp.maximum(m, rows_v[p * _K + j, sl])
                    t = (y1_v[p, sl] + m) * sc_v[sl] + be_v[sl]
                    out_v[p, sl] = jnp.where(
                        t >= jnp.float32(0.0), t, t * jnp.float32(0.2))
            pltpu.sync_copy(out_v, out_hbm.at[pl.ds(pt0, _C)])

    return sck(y2t, idx_flat, y1t, scale, beta)


def kernel(x, W, gamma, beta):
    wm = (W[:, :_D] - W[:, _D:]).T
    w2t = W[:, _D:].T
    idx, y1t, y2t = _tc_stage(x, wm, w2t)
    idx_flat = idx.reshape(_PTS * _K)
    scale = gamma * jnp.float32(1.0 / (1.0 + 1e-5) ** 0.5)
    outt = _sc_stage(y2t.reshape(_PTS, _COUT), idx_flat,
                     y1t.reshape(_PTS, _COUT), scale, beta)
    return outt.reshape(_B, _N, _COUT).transpose(0, 2, 1)


# speedup vs baseline: 10.1131x; 1.2577x over previous
import functools

import jax
import jax.numpy as jnp
from jax import lax
from jax.experimental import pallas as pl
from jax.experimental.pallas import tpu as pltpu
from jax.experimental.pallas import tpu_sc as plsc

_B, _D, _N = 4, 64, 1024
_K = 20
_COUT = 128

_NC, _NS, _L = 2, 16, 16
_NW = _NC * _NS
_PTS = _B * _N
_PER_W = _PTS // _NW
_C = 4
_CH = _PER_W // _C


def _tc_body(x_ref, wm_ref, w2t_ref, idx_ref, y1_ref, y2_ref):
    b = pl.program_id(0)
    xb = x_ref[0]
    xt = xb.T
    g = jnp.dot(xt, xb, preferred_element_type=jnp.float32)
    sqr = jnp.sum(xb * xb, axis=0, keepdims=True)
    sqc = jnp.sum(xt * xt, axis=1, keepdims=True)
    d2 = sqc + sqr - 2.0 * g
    iota = lax.broadcasted_iota(jnp.int32, (_N, _N), 1)
    kiota = lax.broadcasted_iota(jnp.int32, (_N, _K), 1)
    inf = jnp.float32(3.0e38)
    dm = d2
    idx_mat = jnp.zeros((_N, _K), dtype=jnp.int32)
    for j in range(_K):
        rowmin = jnp.min(dm, axis=1, keepdims=True)
        amin = jnp.min(jnp.where(dm == rowmin, iota, _N),
                       axis=1, keepdims=True)
        idx_mat = jnp.where(kiota == j, amin, idx_mat)
        dm = jnp.where(iota == amin, inf, dm)
    idx_ref[0] = idx_mat + b * _N
    y1_ref[0] = jnp.dot(xt, wm_ref[...], preferred_element_type=jnp.float32)
    y2_ref[0] = jnp.dot(xt, w2t_ref[...], preferred_element_type=jnp.float32)


def _tc_stage(x, wm, w2t):
    return pl.pallas_call(
        _tc_body,
        grid=(_B,),
        in_specs=[
            pl.BlockSpec((1, _D, _N), lambda b: (b, 0, 0)),
            pl.BlockSpec((_D, _COUT), lambda b: (0, 0)),
            pl.BlockSpec((_D, _COUT), lambda b: (0, 0)),
        ],
        out_specs=[
            pl.BlockSpec((1, _N, _K), lambda b: (b, 0, 0)),
            pl.BlockSpec((1, _N, _COUT), lambda b: (b, 0, 0)),
            pl.BlockSpec((1, _N, _COUT), lambda b: (b, 0, 0)),
        ],
        out_shape=[
            jax.ShapeDtypeStruct((_B, _N, _K), jnp.int32),
            jax.ShapeDtypeStruct((_B, _N, _COUT), jnp.float32),
            jax.ShapeDtypeStruct((_B, _N, _COUT), jnp.float32),
        ],
    )(x, wm, w2t)


def _sc_stage(y2t, idx_flat, y1t, scale, beta):
    mesh = plsc.VectorSubcoreMesh(core_axis_name="c", subcore_axis_name="s")
    ck = _C * _K

    @functools.partial(
        pl.kernel,
        mesh=mesh,
        out_type=jax.ShapeDtypeStruct((_PTS, _COUT), jnp.float32),
        scratch_types=[
            pltpu.VMEM((_PER_W * _K,), jnp.int32),
            pltpu.VMEM((_PER_W, _COUT), jnp.float32),
            pltpu.VMEM((_PER_W, _COUT), jnp.float32),
            pltpu.VMEM((ck, _COUT), jnp.float32),
            pltpu.VMEM((ck, _COUT), jnp.float32),
            pltpu.VMEM((_COUT,), jnp.float32),
            pltpu.VMEM((_COUT,), jnp.float32),
            pltpu.SemaphoreType.DMA,
            pltpu.SemaphoreType.DMA,
        ],
    )
    def sck(y2t_hbm, idx_hbm, y1t_hbm, sc_hbm, be_hbm, out_hbm,
            idx_all, y1_all, out_all, rows_a, rows_b, sc_v, be_v,
            sem_a, sem_b):
        wid = lax.axis_index("s") * _NC + lax.axis_index("c")
        base = wid * _PER_W
        pltpu.sync_copy(sc_hbm, sc_v)
        pltpu.sync_copy(be_hbm, be_v)
        pltpu.sync_copy(idx_hbm.at[pl.ds(base * _K, _PER_W * _K)], idx_all)
        pltpu.sync_copy(y1t_hbm.at[pl.ds(base, _PER_W)], y1_all)

        def g_start(ci, rows, sem):
            pltpu.make_async_copy(
                y2t_hbm.at[idx_all.at[pl.ds(ci * ck, ck)]], rows, sem).start()

        def g_wait(rows, sem):
            pltpu.make_async_copy(y2t_hbm.at[pl.ds(0, ck)], rows, sem).wait()

        def compute(ci, rows):
            for p in range(_C):
                pp = ci * _C + p
                for g in range(_COUT // _L):
                    sl = pl.ds(g * _L, _L)
                    m = rows[p * _K, sl]
                    for j in range(1, _K):
                        m = jnp.maximum(m, rows[p * _K + j, sl])
                    t = (y1_all[pp, sl] + m) * sc_v[sl] + be_v[sl]
                    out_all[pp, sl] = jnp.where(
                        t >= jnp.float32(0.0), t, t * jnp.float32(0.2))

        g_start(0, rows_a, sem_a)

        @pl.loop(0, _CH // 2)
        def _pair(i):
            ca = 2 * i
            g_start(ca + 1, rows_b, sem_b)
            g_wait(rows_a, sem_a)
            compute(ca, rows_a)

            @pl.when(i < _CH // 2 - 1)
            def _():
                g_start(ca + 2, rows_a, sem_a)

            g_wait(rows_b, sem_b)
            compute(ca + 1, rows_b)

        pltpu.sync_copy(out_all, out_hbm.at[pl.ds(base, _PER_W)])

    return sck(y2t, idx_flat, y1t, scale, beta)


def kernel(x, W, gamma, beta):
    wm = (W[:, :_D] - W[:, _D:]).T
    w2t = W[:, _D:].T
    idx, y1t, y2t = _tc_stage(x, wm, w2t)
    idx_flat = idx.reshape(_PTS * _K)
    scale = gamma * jnp.float32(1.0 / (1.0 + 1e-5) ** 0.5)
    outt = _sc_stage(y2t.reshape(_PTS, _COUT), idx_flat,
                     y1t.reshape(_PTS, _COUT), scale, beta)
    return outt.reshape(_B, _N, _COUT).transpose(0, 2, 1)


# speedup vs baseline: 12.0234x; 1.1889x over previous
import functools

import jax
import jax.numpy as jnp
from jax import lax
from jax.experimental import pallas as pl
from jax.experimental.pallas import tpu as pltpu
from jax.experimental.pallas import tpu_sc as plsc

_B, _D, _N = 4, 64, 1024
_K = 20
_COUT = 128

_NC, _NS, _L = 2, 16, 16
_NW = _NC * _NS
_PTS = _B * _N
_PER_W = _PTS // _NW
_C = 4
_CH = _PER_W // _C


def _tc_body(x_ref, wm_ref, w2t_ref, idx_ref, y1_ref, y2_ref):
    b = pl.program_id(0)
    xb = x_ref[0]
    xt = xb.T
    g = jnp.dot(xt, xb, preferred_element_type=jnp.float32)
    sqr = jnp.sum(xb * xb, axis=0, keepdims=True)
    sqc = jnp.sum(xt * xt, axis=1, keepdims=True)
    d2 = jnp.maximum(sqc + sqr - 2.0 * g, 0.0)
    iota = lax.broadcasted_iota(jnp.int32, (_N, _N), 1)
    kiota = lax.broadcasted_iota(jnp.int32, (_N, _K), 1)
    keys = (lax.bitcast_convert_type(d2, jnp.int32) & ~jnp.int32(1023)) | iota
    imax = jnp.int32(2**31 - 1)
    idx_mat = jnp.zeros((_N, _K), dtype=jnp.int32)
    for j in range(_K):
        rowmin = jnp.min(keys, axis=1, keepdims=True)
        idx_mat = jnp.where(kiota == j, rowmin & 1023, idx_mat)
        keys = jnp.where(keys == rowmin, imax, keys)
    idx_ref[0] = idx_mat + b * _N
    y1_ref[0] = jnp.dot(xt, wm_ref[...], preferred_element_type=jnp.float32)
    y2_ref[0] = jnp.dot(xt, w2t_ref[...], preferred_element_type=jnp.float32)


def _tc_stage(x, wm, w2t):
    return pl.pallas_call(
        _tc_body,
        grid=(_B,),
        in_specs=[
            pl.BlockSpec((1, _D, _N), lambda b: (b, 0, 0)),
            pl.BlockSpec((_D, _COUT), lambda b: (0, 0)),
            pl.BlockSpec((_D, _COUT), lambda b: (0, 0)),
        ],
        out_specs=[
            pl.BlockSpec((1, _N, _K), lambda b: (b, 0, 0)),
            pl.BlockSpec((1, _N, _COUT), lambda b: (b, 0, 0)),
            pl.BlockSpec((1, _N, _COUT), lambda b: (b, 0, 0)),
        ],
        out_shape=[
            jax.ShapeDtypeStruct((_B, _N, _K), jnp.int32),
            jax.ShapeDtypeStruct((_B, _N, _COUT), jnp.float32),
            jax.ShapeDtypeStruct((_B, _N, _COUT), jnp.float32),
        ],
    )(x, wm, w2t)


def _sc_stage(y2t, idx_flat, y1t, scale, beta):
    mesh = plsc.VectorSubcoreMesh(core_axis_name="c", subcore_axis_name="s")
    ck = _C * _K

    @functools.partial(
        pl.kernel,
        mesh=mesh,
        out_type=jax.ShapeDtypeStruct((_PTS, _COUT), jnp.float32),
        scratch_types=[
            pltpu.VMEM((_PER_W * _K,), jnp.int32),
            pltpu.VMEM((_PER_W, _COUT), jnp.float32),
            pltpu.VMEM((_PER_W, _COUT), jnp.float32),
            pltpu.VMEM((ck, _COUT), jnp.float32),
            pltpu.VMEM((ck, _COUT), jnp.float32),
            pltpu.VMEM((_COUT,), jnp.float32),
            pltpu.VMEM((_COUT,), jnp.float32),
            pltpu.SemaphoreType.DMA,
            pltpu.SemaphoreType.DMA,
        ],
    )
    def sck(y2t_hbm, idx_hbm, y1t_hbm, sc_hbm, be_hbm, out_hbm,
            idx_all, y1_all, out_all, rows_a, rows_b, sc_v, be_v,
            sem_a, sem_b):
        wid = lax.axis_index("s") * _NC + lax.axis_index("c")
        base = wid * _PER_W
        pltpu.sync_copy(sc_hbm, sc_v)
        pltpu.sync_copy(be_hbm, be_v)
        pltpu.sync_copy(idx_hbm.at[pl.ds(base * _K, _PER_W * _K)], idx_all)
        pltpu.sync_copy(y1t_hbm.at[pl.ds(base, _PER_W)], y1_all)

        def g_start(ci, rows, sem):
            pltpu.make_async_copy(
                y2t_hbm.at[idx_all.at[pl.ds(ci * ck, ck)]], rows, sem).start()

        def g_wait(rows, sem):
            pltpu.make_async_copy(y2t_hbm.at[pl.ds(0, ck)], rows, sem).wait()

        def compute(ci, rows):
            for p in range(_C):
                pp = ci * _C + p
                for g in range(_COUT // _L):
                    sl = pl.ds(g * _L, _L)
                    m = rows[p * _K, sl]
                    for j in range(1, _K):
                        m = jnp.maximum(m, rows[p * _K + j, sl])
                    t = (y1_all[pp, sl] + m) * sc_v[sl] + be_v[sl]
                    out_all[pp, sl] = jnp.where(
                        t >= jnp.float32(0.0), t, t * jnp.float32(0.2))

        g_start(0, rows_a, sem_a)

        @pl.loop(0, _CH // 2)
        def _pair(i):
            ca = 2 * i
            g_start(ca + 1, rows_b, sem_b)
            g_wait(rows_a, sem_a)
            compute(ca, rows_a)

            @pl.when(i < _CH // 2 - 1)
            def _():
                g_start(ca + 2, rows_a, sem_a)

            g_wait(rows_b, sem_b)
            compute(ca + 1, rows_b)

        pltpu.sync_copy(out_all, out_hbm.at[pl.ds(base, _PER_W)])

    return sck(y2t, idx_flat, y1t, scale, beta)


def kernel(x, W, gamma, beta):
    wm = (W[:, :_D] - W[:, _D:]).T
    w2t = W[:, _D:].T
    idx, y1t, y2t = _tc_stage(x, wm, w2t)
    idx_flat = idx.reshape(_PTS * _K)
    scale = gamma * jnp.float32(1.0 / (1.0 + 1e-5) ** 0.5)
    outt = _sc_stage(y2t.reshape(_PTS, _COUT), idx_flat,
                     y1t.reshape(_PTS, _COUT), scale, beta)
    return outt.reshape(_B, _N, _COUT).transpose(0, 2, 1)
